# baseline (speedup 1.0000x reference)
import numpy as np

import jax
import jax.numpy as jnp
from jax.experimental import pallas as pl
from jax.experimental.pallas import tpu as pltpu


def _routing_constants():
    rdup = np.repeat(np.eye(32, dtype=np.float32), 4, axis=0)
    qdup = np.repeat(np.eye(12, dtype=np.float32), 4, axis=1)
    mask = (np.arange(128)[:, None] % 4 ==
            np.arange(48)[None, :] % 4).astype(np.float32)

    rplace = np.zeros((3, 12, 4), np.float32)
    for c in range(3):
        rplace[c, c * 4:(c + 1) * 4, :] = np.eye(4)

    a2 = np.zeros((8, 24, 12), np.float32)
    bg = np.zeros((8, 16, 64), np.float32)
    for gi, (kh1, kh0, kw2) in enumerate(
            [(a, b, d) for a in range(2) for b in range(2) for d in range(2)]):
        for c in range(3):
            for kh2 in range(2):
                dh = kh2 * 4 + kh1 * 2 + kh0
                a2[gi, c * 8 + dh, c * 4 + kh2 * 2 + kw2] = 1.0
        for kw1 in range(2):
            for kw0 in range(2):
                dw = kw2 * 4 + kw1 * 2 + kw0
                lane16 = kh0 * 8 + kw0 * 4 + kh1 * 2 + kw1
                for wo in range(8):
                    bg[gi, lane16, wo * 8 + dw] = 1.0
    return rdup, qdup, mask, rplace, a2, bg


_RDUP, _QDUP, _MASK, _RPLACE, _A2, _BG = _routing_constants()


def _fused_body(x_ref, w1_ref, b1_ref, w2_ref, b2_ref, w3_ref, b3_ref,
                s_ref, rdup_ref, qdup_ref, mask_ref, rpl_ref, a2_ref, bg_ref,
                o_ref):
    nb = x_ref.shape[0]
    f32 = jnp.float32
    dn = (((0,), (0,)), ((), ()))

    w1r = w1_ref[...]
    w2r = w2_ref[...]
    w3r = w3_ref[...]
    rdup = rdup_ref[...]

    d = jax.lax.dot_general(w3r, w2r, dn, preferred_element_type=f32)
    k = (jnp.dot(rdup, w1r, preferred_element_type=f32)
         @ qdup_ref[...]) * mask_ref[...]
    e = jnp.dot(d, k, preferred_element_type=f32)
    estk = sum(jnp.dot(rpl_ref[12 * c:12 * (c + 1), :],
                       e[:, 16 * c:16 * (c + 1)],
                       preferred_element_type=f32)
               for c in range(3))
    w24 = sum(jnp.dot(jnp.dot(a2_ref[24 * g:24 * (g + 1), :], estk,
                              preferred_element_type=f32),
                      bg_ref[16 * g:16 * (g + 1), :],
                      preferred_element_type=f32)
              for g in range(8))
    w24 = w24.astype(jnp.bfloat16).astype(f32)
    wrow = w24.reshape(3, 8, 64)

    w3s = jnp.sum(w3r, axis=1, keepdims=True)
    s2a = jax.lax.dot_general(w3s, w2r, dn, preferred_element_type=f32)
    s2 = jnp.dot(s2a, rdup, preferred_element_type=f32)
    bb1 = jnp.sum(s2 * b1_ref[...], axis=1, keepdims=True)
    bb2 = jnp.dot(b2_ref[...], w3s, preferred_element_type=f32)
    bfull = bb1 + bb2 + b3_ref[...]

    x5 = x_ref[...].reshape(nb, 3, 8, 8, 64)
    x5 = x5.astype(jnp.bfloat16).astype(f32)
    s = jnp.sum(x5 * wrow[None, :, None, :, :], axis=(1, 3))
    r = jnp.dot(s.reshape(nb * 8, 64), s_ref[...], preferred_element_type=f32)
    o_ref[...] = r + bfull


def kernel(conv_1_w, conv_1_b, conv_2_w, conv_2_b, conv_3_w, conv_3_b, x):
    N, C, H, W = x.shape
    Ho, Wo = H // 8, W // 8

    w1f = conv_1_w.reshape(32, 12)
    w2f = conv_2_w.reshape(64, 128)
    w3f = conv_3_w.reshape(64, 4)
    b1f = conv_1_b.reshape(1, 32)
    b2f = conv_2_b.reshape(1, 64)
    b3f = conv_3_b.reshape(1, 1)

    S = (jnp.arange(W)[:, None] // 8 ==
         jnp.arange(Wo)[None, :]).astype(jnp.float32)

    rdup = jnp.asarray(_RDUP)
    qdup = jnp.asarray(_QDUP)
    mask = jnp.asarray(_MASK)
    rpl = jnp.asarray(_RPLACE.reshape(36, 4))
    a2 = jnp.asarray(_A2.reshape(192, 12))
    bg = jnp.asarray(_BG.reshape(128, 64))

    NB = 128
    while N % NB:
        NB //= 2
    out = pl.pallas_call(
        _fused_body,
        out_shape=jax.ShapeDtypeStruct((N * Ho, Wo), jnp.float32),
        grid_spec=pltpu.PrefetchScalarGridSpec(
            num_scalar_prefetch=0,
            grid=(N // NB,),
            in_specs=[
                pl.BlockSpec((NB, C, H, W), lambda i: (i, 0, 0, 0)),
                pl.BlockSpec((32, 12), lambda i: (0, 0)),
                pl.BlockSpec((1, 32), lambda i: (0, 0)),
                pl.BlockSpec((64, 128), lambda i: (0, 0)),
                pl.BlockSpec((1, 64), lambda i: (0, 0)),
                pl.BlockSpec((64, 4), lambda i: (0, 0)),
                pl.BlockSpec((1, 1), lambda i: (0, 0)),
                pl.BlockSpec((W, Wo), lambda i: (0, 0)),
                pl.BlockSpec((128, 32), lambda i: (0, 0)),
                pl.BlockSpec((12, 48), lambda i: (0, 0)),
                pl.BlockSpec((128, 48), lambda i: (0, 0)),
                pl.BlockSpec((36, 4), lambda i: (0, 0)),
                pl.BlockSpec((192, 12), lambda i: (0, 0)),
                pl.BlockSpec((128, 64), lambda i: (0, 0)),
            ],
            out_specs=pl.BlockSpec((NB * Ho, Wo), lambda i: (i, 0)),
        ),
        compiler_params=pltpu.CompilerParams(
            dimension_semantics=("parallel",),
            vmem_limit_bytes=64 * 1024 * 1024,
        ),
    )(x, w1f, b1f, w2f, b2f, w3f, b3f, S, rdup, qdup, mask, rpl, a2, bg)

    return out.reshape(N, 1, Ho, Wo).astype(x.dtype)


# speedup vs baseline: 1.2396x; 1.2396x over previous
import jax
import jax.numpy as jnp
from jax.experimental import pallas as pl
from jax.experimental.pallas import tpu as pltpu


def _fused_body(x_ref, w_ref, s_ref, b_ref, o_ref):
    nb = x_ref.shape[0] // 192
    x5 = x_ref[...].reshape(nb, 3, 8, 8, 64)
    x5 = x5.astype(jnp.bfloat16).astype(jnp.float32)
    s = jnp.sum(x5 * w_ref[...][None, :, None, :, :], axis=(1, 3))
    r = jnp.dot(s.reshape(nb * 8, 64), s_ref[...],
                preferred_element_type=jnp.float32)
    o_ref[...] = r + b_ref[...]


def kernel(conv_1_w, conv_1_b, conv_2_w, conv_2_b, conv_3_w, conv_3_b, x):
    N, C, H, W = x.shape
    Ho, Wo = H // 8, W // 8

    t = jnp.einsum("uckl,cvij->uvklij", conv_2_w, conv_1_w)
    wfull = jnp.einsum("upq,uvklij->vpkiqlj", conv_3_w[0], t).reshape(C, 8, 8)
    wfull = wfull.astype(jnp.bfloat16).astype(jnp.float32)
    wrow = jnp.tile(wfull, (1, 1, Wo))

    w3s = conv_3_w[0].sum(axis=(1, 2))
    bfull = (jnp.einsum("c,uckl,u->", conv_1_b, conv_2_w, w3s)
             + conv_2_b @ w3s + conv_3_b[0]).reshape(1, 1).astype(jnp.float32)

    S = (jnp.arange(W)[:, None] // 8 ==
         jnp.arange(Wo)[None, :]).astype(jnp.float32)

    xf = x.reshape(N * C * H, W)

    NB = 128
    while N % NB:
        NB //= 2
    out = pl.pallas_call(
        _fused_body,
        out_shape=jax.ShapeDtypeStruct((N * Ho, Wo), jnp.float32),
        grid_spec=pltpu.PrefetchScalarGridSpec(
            num_scalar_prefetch=0,
            grid=(N // NB,),
            in_specs=[
                pl.BlockSpec((NB * C * H, W), lambda i: (i, 0)),
                pl.BlockSpec((C, 8, W), lambda i: (0, 0, 0)),
                pl.BlockSpec((W, Wo), lambda i: (0, 0)),
                pl.BlockSpec((1, 1), lambda i: (0, 0)),
            ],
            out_specs=pl.BlockSpec((NB * Ho, Wo), lambda i: (i, 0)),
        ),
        compiler_params=pltpu.CompilerParams(
            dimension_semantics=("parallel",),
            vmem_limit_bytes=64 * 1024 * 1024,
        ),
    )(xf, wrow, S, bfull)

    return out.reshape(N, 1, Ho, Wo).astype(x.dtype)
